# baseline (speedup 1.0000x reference)
import functools

import jax
import jax.numpy as jnp
from jax import lax
from jax.experimental import pallas as pl
from jax.experimental.pallas import tpu as pltpu
from jax.experimental.pallas import tpu_sc as plsc

B = 16384
D = 64

NC = 2
NS = 16
NW = NC * NS
B_PER_W = B // NW
CHUNK = 128
NCHUNK = B_PER_W // CHUNK


def _gather_body(table_hbm, idx_q_hbm, idx_a_hbm, q_out_hbm, a_out_hbm,
                 idx_q_v, idx_a_v, rows_q_v, rows_a_v, sem):
    wid = lax.axis_index("s") * NC + lax.axis_index("c")
    base = wid * B_PER_W

    pltpu.sync_copy(idx_q_hbm.at[pl.ds(base, B_PER_W)], idx_q_v)
    pltpu.sync_copy(idx_a_hbm.at[pl.ds(base, B_PER_W)], idx_a_v)

    copies = []
    for j in range(NCHUNK):
        sl = pl.ds(j * CHUNK, CHUNK)
        copies.append(pltpu.async_copy(
            table_hbm.at[idx_q_v.at[sl]], rows_q_v.at[sl], sem))
        copies.append(pltpu.async_copy(
            table_hbm.at[idx_a_v.at[sl]], rows_a_v.at[sl], sem))
    for c in copies:
        c.wait()

    pltpu.sync_copy(rows_q_v, q_out_hbm.at[pl.ds(base, B_PER_W)])
    pltpu.sync_copy(rows_a_v, a_out_hbm.at[pl.ds(base, B_PER_W)])


@functools.partial(jax.jit, static_argnums=())
def _gather2(table, idx_q, idx_a):
    run = functools.partial(
        pl.kernel,
        out_type=(
            jax.ShapeDtypeStruct((B, D), jnp.float32),
            jax.ShapeDtypeStruct((B, D), jnp.float32),
        ),
        mesh=plsc.VectorSubcoreMesh(core_axis_name="c", subcore_axis_name="s"),
        scratch_types=[
            pltpu.VMEM((B_PER_W,), jnp.int32),
            pltpu.VMEM((B_PER_W,), jnp.int32),
            pltpu.VMEM((B_PER_W, D), jnp.float32),
            pltpu.VMEM((B_PER_W, D), jnp.float32),
            pltpu.SemaphoreType.DMA,
        ],
        compiler_params=pltpu.CompilerParams(use_tc_tiling_on_sc=False),
    )(_gather_body)
    return run(table, idx_q, idx_a)


def kernel(cat_q, num_q, cat_qu, num_qu, cat_au, num_au, emb_user):
    idx_q = cat_qu.reshape(B)
    idx_a = cat_au.reshape(B)
    return _gather2(emb_user, idx_q, idx_a)
